# baseline (speedup 1.0000x reference)
import functools

import jax
import jax.numpy as jnp
from jax import lax
from jax.experimental import pallas as pl
from jax.experimental.pallas import tpu as pltpu
from jax.experimental.pallas import tpu_sc as plsc

_L = 16
_NW = 32
_D = 64
_CH = _D // _L
_S = 4096
_T = 208
_SPW = _S // _NW
_RPI = 8

_mesh = plsc.VectorSubcoreMesh(core_axis_name="c", subcore_axis_name="s")


@functools.partial(
    pl.kernel,
    out_type=jax.ShapeDtypeStruct((_S, 2), jnp.float32),
    mesh=_mesh,
    scratch_types=[
        pltpu.VMEM((_SPW * _T,), jnp.int32),
        pltpu.VMEM((_T, _D), jnp.float32),
        pltpu.VMEM((_T, _D), jnp.float32),
        pltpu.VMEM((192,), jnp.float32),
        pltpu.VMEM((_SPW, 2), jnp.float32),
        pltpu.SemaphoreType.DMA,
        pltpu.SemaphoreType.DMA,
    ],
    compiler_params=pltpu.CompilerParams(
        needs_layout_passes=False, use_tc_tiling_on_sc=False),
)
def _fasttext_sc(sent_hbm, emb_hbm, pw_hbm, out_hbm,
                 idx_v, rows_a, rows_b, pw_v, out_v, sem_a, sem_b):
  wid = lax.axis_index("s") * 2 + lax.axis_index("c")
  base = wid * _SPW

  pltpu.sync_copy(sent_hbm.at[pl.ds(base * _T, _SPW * _T)], idx_v)
  pltpu.sync_copy(pw_hbm, pw_v)

  w_vecs = [[pw_v[pl.ds(k * _D + c * _L, _L)] for c in range(_CH)]
            for k in range(2)]
  bias_v = pw_v[pl.ds(2 * _D, _L)]
  e0w_v = pw_v[pl.ds(2 * _D + 8, _L)]
  lane = jax.lax.broadcasted_iota(jnp.int32, (_L,), 0)

  def start_gather(j, rows, sem):
    pltpu.async_copy(
        emb_hbm.at[idx_v.at[pl.ds(j * _T, _T // 2)]],
        rows.at[pl.ds(0, _T // 2)], sem)
    pltpu.async_copy(
        emb_hbm.at[idx_v.at[pl.ds(j * _T + _T // 2, _T // 2)]],
        rows.at[pl.ds(_T // 2, _T // 2)], sem)

  def wait_gather(rows, sem):
    pltpu.make_async_copy(emb_hbm.at[pl.ds(0, _T)], rows, sem).wait()

  def compute(j, rows):
    zc = jnp.zeros((_L,), jnp.float32)
    for k in range(_T // _L):
      tok = idx_v[pl.ds(j * _T + k * _L, _L)]
      zc = zc + jnp.where(tok == 0, 1.0, 0.0).astype(jnp.float32)
    n_pad = jnp.sum(zc)

    def row_body(i, accs):
      t = i * _RPI
      new = list(accs)
      for r in range(_RPI):
        for c in range(_CH):
          new[c] = new[c] + rows[t + r, pl.ds(c * _L, _L)]
      return tuple(new)

    accs = tuple(jnp.zeros((_L,), jnp.float32) for _ in range(_CH))
    accs = lax.fori_loop(0, _T // _RPI, row_body, accs)

    raw = []
    for k in range(2):
      dot = accs[0] * w_vecs[k][0]
      for c in range(1, _CH):
        dot = dot + accs[c] * w_vecs[k][c]
      raw.append(jnp.sum(dot))
    raw_v = jnp.where(lane == 0, raw[0], raw[1])
    n_pad_v = jnp.full((_L,), n_pad)
    vals = (raw_v - n_pad_v * e0w_v) / (float(_T) - n_pad_v) + bias_v
    plsc.store_scatter(out_v, [jnp.full((_L,), j, jnp.int32), lane], vals,
                       mask=lane < 2)

  start_gather(0, rows_a, sem_a)

  def body(i, carry):
    j0 = 2 * i
    start_gather(j0 + 1, rows_b, sem_b)
    wait_gather(rows_a, sem_a)
    compute(j0, rows_a)

    @pl.when(i < _SPW // 2 - 1)
    def _():
      start_gather(j0 + 2, rows_a, sem_a)

    wait_gather(rows_b, sem_b)
    compute(j0 + 1, rows_b)
    return carry

  lax.fori_loop(0, _SPW // 2, body, 0)

  pltpu.sync_copy(out_v, out_hbm.at[pl.ds(base, _SPW)])


def kernel(sentence, emb, W, b):
  sent = jnp.pad(sentence.astype(jnp.int32), ((0, 0), (0, _T - 200)))
  e0w = emb[0] @ W.T
  pw = jnp.concatenate(
      [W.reshape(-1), b, jnp.zeros((6,), jnp.float32),
       e0w, jnp.zeros((54,), jnp.float32)])
  return _fasttext_sc(sent.reshape(-1), emb, pw)


# speedup vs baseline: 1.0009x; 1.0009x over previous
import functools

import jax
import jax.numpy as jnp
from jax import lax
from jax.experimental import pallas as pl
from jax.experimental.pallas import tpu as pltpu
from jax.experimental.pallas import tpu_sc as plsc

_L = 16
_NW = 32
_D = 64
_CH = _D // _L
_S = 4096
_T = 208
_SPW = _S // _NW
_RPI = 8

_mesh = plsc.VectorSubcoreMesh(core_axis_name="c", subcore_axis_name="s")


@functools.partial(
    pl.kernel,
    out_type=jax.ShapeDtypeStruct((_S, 2), jnp.float32),
    mesh=_mesh,
    scratch_types=[
        pltpu.VMEM((_SPW * _T,), jnp.int32),
        pltpu.VMEM((_T, _D), jnp.float32),
        pltpu.VMEM((_T, _D), jnp.float32),
        pltpu.VMEM((_T, _D), jnp.float32),
        pltpu.VMEM((_T, _D), jnp.float32),
        pltpu.VMEM((192,), jnp.float32),
        pltpu.VMEM((_SPW, 2), jnp.float32),
        pltpu.SemaphoreType.DMA,
        pltpu.SemaphoreType.DMA,
        pltpu.SemaphoreType.DMA,
        pltpu.SemaphoreType.DMA,
    ],
    compiler_params=pltpu.CompilerParams(
        needs_layout_passes=False, use_tc_tiling_on_sc=False),
)
def _fasttext_sc(sent_hbm, emb_hbm, pw_hbm, out_hbm,
                 idx_v, rb0, rb1, rb2, rb3, pw_v, out_v,
                 sm0, sm1, sm2, sm3):
  wid = lax.axis_index("s") * 2 + lax.axis_index("c")
  base = wid * _SPW
  bufs = [rb0, rb1, rb2, rb3]
  sems = [sm0, sm1, sm2, sm3]
  nbuf = len(bufs)

  pltpu.sync_copy(sent_hbm.at[pl.ds(base * _T, _SPW * _T)], idx_v)
  pltpu.sync_copy(pw_hbm, pw_v)

  w_vecs = [[pw_v[pl.ds(k * _D + c * _L, _L)] for c in range(_CH)]
            for k in range(2)]
  bias_v = pw_v[pl.ds(2 * _D, _L)]
  e0w_v = pw_v[pl.ds(2 * _D + 8, _L)]
  lane = jax.lax.broadcasted_iota(jnp.int32, (_L,), 0)

  def start_gather(j, rows, sem):
    pltpu.async_copy(emb_hbm.at[idx_v.at[pl.ds(j * _T, _T)]], rows, sem)

  def wait_gather(j, rows, sem):
    pltpu.make_async_copy(emb_hbm.at[idx_v.at[pl.ds(j * _T, _T)]], rows, sem).wait()

  def compute(j, rows):
    def row_body(i, carry):
      accs, zc = carry
      tok = idx_v[pl.ds(j * _T + i * _L, _L)]
      zc = zc + jnp.where(tok == 0, 1.0, 0.0).astype(jnp.float32)
      new = list(accs)
      for r in range(_L):
        for c in range(_CH):
          new[c] = new[c] + rows[i * _L + r, pl.ds(c * _L, _L)]
      return tuple(new), zc

    accs = tuple(jnp.zeros((_L,), jnp.float32) for _ in range(_CH))
    zc = jnp.zeros((_L,), jnp.float32)
    accs, zc = lax.fori_loop(0, _T // _L, row_body, (accs, zc))
    n_pad = jnp.sum(zc)

    raw = []
    for k in range(2):
      dot = accs[0] * w_vecs[k][0]
      for c in range(1, _CH):
        dot = dot + accs[c] * w_vecs[k][c]
      raw.append(jnp.sum(dot))
    raw_v = jnp.where(lane == 0, raw[0], raw[1])
    n_pad_v = jnp.full((_L,), n_pad)
    vals = (raw_v - n_pad_v * e0w_v) / (float(_T) - n_pad_v) + bias_v
    plsc.store_scatter(out_v, [jnp.full((_L,), j, jnp.int32), lane], vals,
                       mask=lane < 2)

  for b in range(nbuf):
    start_gather(b, bufs[b], sems[b])

  def body(g, carry):
    for b in range(nbuf):
      j = g * nbuf + b
      wait_gather(j, bufs[b], sems[b])
      compute(j, bufs[b])

      @pl.when(j + nbuf < _SPW)
      def _():
        start_gather(j + nbuf, bufs[b], sems[b])
    return carry

  lax.fori_loop(0, _SPW // nbuf, body, 0)

  pltpu.sync_copy(out_v, out_hbm.at[pl.ds(base, _SPW)])


def kernel(sentence, emb, W, b):
  sent = jnp.pad(sentence.astype(jnp.int32), ((0, 0), (0, _T - 200)))
  e0w = emb[0] @ W.T
  pw = jnp.concatenate(
      [W.reshape(-1), b, jnp.zeros((6,), jnp.float32),
       e0w, jnp.zeros((54,), jnp.float32)])
  return _fasttext_sc(sent.reshape(-1), emb, pw)


# speedup vs baseline: 1.6833x; 1.6818x over previous
import functools

import jax
import jax.numpy as jnp
from jax import lax
from jax.experimental import pallas as pl
from jax.experimental.pallas import tpu as pltpu
from jax.experimental.pallas import tpu_sc as plsc

_L = 16
_NW = 32
_D = 64
_S = 4096
_T = 208
_SPW = _S // _NW
_V = 1000000
_CHUNK = _V // 8
_C = 1000
_GRID = _CHUNK // _C

_mesh = plsc.VectorSubcoreMesh(core_axis_name="c", subcore_axis_name="s")


def _proj_tc_body(*refs):
  emb_refs, wp_ref, out_ref = refs[:8], refs[8], refs[9]
  wp = wp_ref[...]
  for u in range(8):
    out_ref[:, u * _L:(u + 1) * _L] = jnp.dot(
        emb_refs[u][...], wp, preferred_element_type=jnp.float32)


_proj_tc = pl.pallas_call(
    _proj_tc_body,
    grid=(_GRID,),
    in_specs=[
        pl.BlockSpec((_C, _D), lambda g, u=u: (u * _GRID + g, 0))
        for u in range(8)
    ] + [pl.BlockSpec((_D, _L), lambda g: (0, 0))],
    out_specs=pl.BlockSpec((_C, 8 * _L), lambda g: (g, 0)),
    out_shape=jax.ShapeDtypeStruct((_V // 8, 8 * _L), jnp.float32),
)


@functools.partial(
    pl.kernel,
    out_type=jax.ShapeDtypeStruct((_S, 2), jnp.float32),
    mesh=_mesh,
    scratch_types=[
        pltpu.VMEM((_SPW * _T,), jnp.int32),
        pltpu.VMEM((_T, _L), jnp.float32),
        pltpu.VMEM((_T, _L), jnp.float32),
        pltpu.VMEM((_T, _L), jnp.float32),
        pltpu.VMEM((_T, _L), jnp.float32),
        pltpu.VMEM((1, _L), jnp.float32),
        pltpu.VMEM((_L,), jnp.float32),
        pltpu.VMEM((_SPW, 2), jnp.float32),
        pltpu.SemaphoreType.DMA,
        pltpu.SemaphoreType.DMA,
        pltpu.SemaphoreType.DMA,
        pltpu.SemaphoreType.DMA,
    ],
    compiler_params=pltpu.CompilerParams(
        needs_layout_passes=False, use_tc_tiling_on_sc=False),
)
def _pool_sc(sent_hbm, p_hbm, b_hbm, out_hbm,
             idx_v, rb0, rb1, rb2, rb3, p0_v, b_v, out_v,
             sm0, sm1, sm2, sm3):
  wid = lax.axis_index("s") * 2 + lax.axis_index("c")
  base = wid * _SPW
  bufs = [rb0, rb1, rb2, rb3]
  sems = [sm0, sm1, sm2, sm3]
  nbuf = len(bufs)

  pltpu.sync_copy(sent_hbm.at[pl.ds(base * _T, _SPW * _T)], idx_v)
  pltpu.sync_copy(p_hbm.at[pl.ds(0, 1)], p0_v)
  pltpu.sync_copy(b_hbm, b_v)

  p0 = p0_v[0, :]
  bias_v = b_v[...]
  lane = jax.lax.broadcasted_iota(jnp.int32, (_L,), 0)

  def remap(j):
    def rbody(i, carry):
      off = j * _T + i * _L
      tok = idx_v[pl.ds(off, _L)]
      u = jnp.zeros((_L,), jnp.int32)
      for t in range(1, 8):
        u = u + jnp.where(tok >= t * _CHUNK, 1, 0).astype(jnp.int32)
      idx_v[pl.ds(off, _L)] = (tok - u * _CHUNK) * 8 + u
      return carry

    lax.fori_loop(0, _T // _L, rbody, 0)

  def start_gather(j, rows, sem):
    pltpu.async_copy(p_hbm.at[idx_v.at[pl.ds(j * _T, _T)]], rows, sem)

  def wait_gather(j, rows, sem):
    pltpu.make_async_copy(p_hbm.at[idx_v.at[pl.ds(j * _T, _T)]], rows,
                          sem).wait()

  def compute(j, rows):
    def row_body(i, carry):
      acc, zc = carry
      tok = idx_v[pl.ds(j * _T + i * _L, _L)]
      zc = zc + jnp.where(tok == 0, 1.0, 0.0).astype(jnp.float32)
      for r in range(_L):
        acc = acc + rows[i * _L + r, :]
      return acc, zc

    acc = jnp.zeros((_L,), jnp.float32)
    zc = jnp.zeros((_L,), jnp.float32)
    acc, zc = lax.fori_loop(0, _T // _L, row_body, (acc, zc))
    n_pad = jnp.sum(zc)

    n_pad_v = jnp.full((_L,), n_pad)
    vals = (acc - n_pad_v * p0) / (float(_T) - n_pad_v) + bias_v
    plsc.store_scatter(out_v, [jnp.full((_L,), j, jnp.int32), lane], vals,
                       mask=lane < 2)

  for b in range(nbuf):
    remap(b)
    start_gather(b, bufs[b], sems[b])

  def body(g, carry):
    for b in range(nbuf):
      j = g * nbuf + b

      @pl.when(j + nbuf < _SPW)
      def _():
        remap(j + nbuf)

      wait_gather(j, bufs[b], sems[b])
      compute(j, bufs[b])

      @pl.when(j + nbuf < _SPW)
      def _():
        start_gather(j + nbuf, bufs[b], sems[b])
    return carry

  lax.fori_loop(0, _SPW // nbuf, body, 0)

  pltpu.sync_copy(out_v, out_hbm.at[pl.ds(base, _SPW)])


def kernel(sentence, emb, W, b):
  sent = jnp.pad(sentence.astype(jnp.int32), ((0, 0), (0, _T - 200)))
  wp = jnp.zeros((_D, _L), jnp.float32).at[:, 0:2].set(W.T)
  p = _proj_tc(*([emb] * 8), wp).reshape(_V, _L)
  b16 = jnp.concatenate([b, jnp.zeros((_L - 2,), jnp.float32)])
  return _pool_sc(sent.reshape(-1), p, b16)


# speedup vs baseline: 1.8050x; 1.0723x over previous
import functools

import jax
import jax.numpy as jnp
from jax import lax
from jax.experimental import pallas as pl
from jax.experimental.pallas import tpu as pltpu
from jax.experimental.pallas import tpu_sc as plsc

_L = 16
_NW = 32
_D = 64
_S = 4096
_T = 208
_SPW = _S // _NW
_V = 1000000
_CHUNK = _V // 8
_C = 5000
_GRID = _CHUNK // _C

_mesh = plsc.VectorSubcoreMesh(core_axis_name="c", subcore_axis_name="s")


def _proj_tc_body(*refs):
  emb_refs, wp_ref, out_ref = refs[:8], refs[8], refs[9]
  wp = wp_ref[...]
  for u in range(8):
    out_ref[:, u * _L:(u + 1) * _L] = jnp.dot(
        emb_refs[u][...], wp, preferred_element_type=jnp.float32)


_proj_tc = pl.pallas_call(
    _proj_tc_body,
    grid=(_GRID,),
    in_specs=[
        pl.BlockSpec((_C, _D), lambda g, u=u: (u * _GRID + g, 0))
        for u in range(8)
    ] + [pl.BlockSpec((_D, _L), lambda g: (0, 0))],
    out_specs=pl.BlockSpec((_C, 8 * _L), lambda g: (g, 0)),
    out_shape=jax.ShapeDtypeStruct((_V // 8, 8 * _L), jnp.float32),
)


@functools.partial(
    pl.kernel,
    out_type=jax.ShapeDtypeStruct((_S, 2), jnp.float32),
    mesh=_mesh,
    scratch_types=[
        pltpu.VMEM((_SPW * _T,), jnp.int32),
        pltpu.VMEM((_T, _L), jnp.float32),
        pltpu.VMEM((_T, _L), jnp.float32),
        pltpu.VMEM((_T, _L), jnp.float32),
        pltpu.VMEM((_T, _L), jnp.float32),
        pltpu.VMEM((1, _L), jnp.float32),
        pltpu.VMEM((_L,), jnp.float32),
        pltpu.VMEM((_SPW, 2), jnp.float32),
        pltpu.SemaphoreType.DMA,
        pltpu.SemaphoreType.DMA,
        pltpu.SemaphoreType.DMA,
        pltpu.SemaphoreType.DMA,
    ],
    compiler_params=pltpu.CompilerParams(
        needs_layout_passes=False, use_tc_tiling_on_sc=False),
)
def _pool_sc(sent_hbm, p_hbm, b_hbm, out_hbm,
             idx_v, rb0, rb1, rb2, rb3, p0_v, b_v, out_v,
             sm0, sm1, sm2, sm3):
  wid = lax.axis_index("s") * 2 + lax.axis_index("c")
  base = wid * _SPW
  bufs = [rb0, rb1, rb2, rb3]
  sems = [sm0, sm1, sm2, sm3]
  nbuf = len(bufs)

  pltpu.sync_copy(sent_hbm.at[pl.ds(base * _T, _SPW * _T)], idx_v)
  pltpu.sync_copy(p_hbm.at[pl.ds(0, 1)], p0_v)
  pltpu.sync_copy(b_hbm, b_v)

  p0 = p0_v[0, :]
  bias_v = b_v[...]
  lane = jax.lax.broadcasted_iota(jnp.int32, (_L,), 0)

  def remap(j):
    def rbody(i, carry):
      off = j * _T + i * _L
      tok = idx_v[pl.ds(off, _L)]
      u = jnp.zeros((_L,), jnp.int32)
      for t in range(1, 8):
        u = u + jnp.where(tok >= t * _CHUNK, 1, 0).astype(jnp.int32)
      idx_v[pl.ds(off, _L)] = (tok - u * _CHUNK) * 8 + u
      return carry

    lax.fori_loop(0, _T // _L, rbody, 0)

  def start_gather(j, rows, sem):
    pltpu.async_copy(p_hbm.at[idx_v.at[pl.ds(j * _T, _T)]], rows, sem)

  def wait_gather(j, rows, sem):
    pltpu.make_async_copy(p_hbm.at[idx_v.at[pl.ds(j * _T, _T)]], rows,
                          sem).wait()

  def compute(j, rows):
    def row_body(i, carry):
      acc, zc = carry
      tok = idx_v[pl.ds(j * _T + i * _L, _L)]
      zc = zc + jnp.where(tok == 0, 1.0, 0.0).astype(jnp.float32)
      for r in range(_L):
        acc = acc + rows[i * _L + r, :]
      return acc, zc

    acc = jnp.zeros((_L,), jnp.float32)
    zc = jnp.zeros((_L,), jnp.float32)
    acc, zc = lax.fori_loop(0, _T // _L, row_body, (acc, zc))
    n_pad = jnp.sum(zc)

    n_pad_v = jnp.full((_L,), n_pad)
    vals = (acc - n_pad_v * p0) / (float(_T) - n_pad_v) + bias_v
    plsc.store_scatter(out_v, [jnp.full((_L,), j, jnp.int32), lane], vals,
                       mask=lane < 2)

  for b in range(nbuf):
    remap(b)
    start_gather(b, bufs[b], sems[b])

  def body(g, carry):
    for b in range(nbuf):
      j = g * nbuf + b

      @pl.when(j + nbuf < _SPW)
      def _():
        remap(j + nbuf)

      wait_gather(j, bufs[b], sems[b])
      compute(j, bufs[b])

      @pl.when(j + nbuf < _SPW)
      def _():
        start_gather(j + nbuf, bufs[b], sems[b])
    return carry

  lax.fori_loop(0, _SPW // nbuf, body, 0)

  pltpu.sync_copy(out_v, out_hbm.at[pl.ds(base, _SPW)])


def kernel(sentence, emb, W, b):
  sent = jnp.pad(sentence.astype(jnp.int32), ((0, 0), (0, _T - 200)))
  wp = jnp.zeros((_D, _L), jnp.float32).at[:, 0:2].set(W.T)
  p = _proj_tc(*([emb] * 8), wp).reshape(_V, _L)
  b16 = jnp.concatenate([b, jnp.zeros((_L - 2,), jnp.float32)])
  return _pool_sc(sent.reshape(-1), p, b16)
